# baseline (speedup 1.0000x reference)
import functools

import jax
import jax.numpy as jnp
from jax import lax
from jax.experimental import pallas as pl
from jax.experimental.pallas import tpu as pltpu
from jax.experimental.pallas import tpu_sc as plsc

_NC = 2
_NS = 16
_NW = _NC * _NS
_CHUNK = 128


def _f32(shape):
    return jax.ShapeDtypeStruct(shape, jnp.float32)


def _off(base, i):
    return pl.ds(pl.multiple_of(base + i * _CHUNK, 8), _CHUNK)


def _make_deg_kernel(K, N, W=128):
    npad = ((N + 8 * _NS - 1) // (8 * _NS)) * (8 * _NS)
    rpt = npad // _NS
    mesh = plsc.VectorSubcoreMesh(core_axis_name="c", subcore_axis_name="s")

    assert K % 2 == 0

    @functools.partial(
        pl.kernel,
        out_type=_f32((_NC, npad, W)),
        mesh=mesh,
        scratch_types=[
            pltpu.VMEM((_CHUNK,), jnp.int32),
            pltpu.VMEM((_CHUNK,), jnp.int32),
            pltpu.VMEM((_CHUNK, W), jnp.float32),
            pltpu.VMEM_SHARED((npad, W), jnp.float32),
            pltpu.SemaphoreType.DMA,
            pltpu.SemaphoreType.DMA,
        ],
    )
    def deg_kernel(dst_h, ones_h, zrows_h, out_h, di0, di1, ones_v, acc,
                   ss0, ss1):
        cid = lax.axis_index("c")
        sid = lax.axis_index("s")
        wid = sid * _NC + cid
        base = wid * (K * _CHUNK)
        pltpu.sync_copy(ones_h, ones_v)
        pltpu.sync_copy(zrows_h, acc.at[pl.ds(sid * rpt, rpt)])
        plsc.subcore_barrier()

        pltpu.sync_copy(dst_h.at[_off(base, 0)], di0)
        pltpu.async_copy(ones_v, acc.at[di0], ss0, add=True)

        def pair(j, carry):
            i0 = 2 * j
            pltpu.sync_copy(dst_h.at[_off(base, i0 + 1)], di1)
            pltpu.async_copy(ones_v, acc.at[di1], ss1, add=True)
            pltpu.make_async_copy(ones_v, acc.at[di0], ss0).wait()

            @pl.when(j < K // 2 - 1)
            def _():
                pltpu.sync_copy(dst_h.at[_off(base, i0 + 2)], di0)
                pltpu.async_copy(ones_v, acc.at[di0], ss0, add=True)

            pltpu.make_async_copy(ones_v, acc.at[di1], ss1).wait()
            return carry

        lax.fori_loop(0, K // 2, pair, 0)
        plsc.subcore_barrier()
        pltpu.sync_copy(
            acc.at[pl.ds(sid * rpt, rpt)],
            out_h.at[cid, pl.ds(sid * rpt, rpt)],
        )

    return deg_kernel


def _make_agg_kernel(K, N, D):
    npad = ((N + 8 * _NS - 1) // (8 * _NS)) * (8 * _NS)
    rpt = npad // _NS
    mesh = plsc.VectorSubcoreMesh(core_axis_name="c", subcore_axis_name="s")
    assert K % 2 == 0

    @functools.partial(
        pl.kernel,
        out_type=_f32((_NC, npad, D)),
        mesh=mesh,
        scratch_types=[
            pltpu.VMEM((_CHUNK,), jnp.int32),
            pltpu.VMEM((_CHUNK,), jnp.int32),
            pltpu.VMEM((_CHUNK,), jnp.int32),
            pltpu.VMEM((_CHUNK,), jnp.int32),
            pltpu.VMEM((_CHUNK, D), jnp.float32),
            pltpu.VMEM((_CHUNK, D), jnp.float32),
            pltpu.VMEM_SHARED((npad, D), jnp.float32),
            pltpu.SemaphoreType.DMA,
            pltpu.SemaphoreType.DMA,
        ],
    )
    def agg_kernel(table_h, src_h, dst_h, zrows_h, out_h,
                   si0, si1, di0, di1, rows0, rows1, acc, sg0, sg1):
        cid = lax.axis_index("c")
        sid = lax.axis_index("s")
        wid = sid * _NC + cid
        base = wid * (K * _CHUNK)
        pltpu.sync_copy(zrows_h, acc.at[pl.ds(sid * rpt, rpt)])
        plsc.subcore_barrier()

        def chunk(i, carry):
            pltpu.sync_copy(src_h.at[_off(base, i)], si0)
            pltpu.sync_copy(dst_h.at[_off(base, i)], di0)
            pltpu.async_copy(table_h.at[si0], rows0, sg0).wait()
            pltpu.sync_copy(rows0, acc.at[di0], add=True)
            return carry

        lax.fori_loop(0, K, chunk, 0)
        plsc.subcore_barrier()
        pltpu.sync_copy(
            acc.at[pl.ds(sid * rpt, rpt)],
            out_h.at[cid, pl.ds(sid * rpt, rpt)],
        )

    return agg_kernel


def _bn_relu(conv, g, be):
    m = jnp.mean(conv, axis=0)
    v = jnp.var(conv, axis=0)
    return jnp.maximum((conv - m) / jnp.sqrt(v + 1e-5) * g + be, 0.0)


def _tc_a(degp, x, W0):
    N, D = x.shape

    def body(degp_r, x_r, w_r, dinv_r, u0_r, t0_r):
        deg = degp_r[0] + degp_r[1] + 1.0
        dinv = lax.rsqrt(deg)
        u0 = jnp.dot(x_r[...], w_r[...], preferred_element_type=jnp.float32)
        dinv_r[...] = dinv
        u0_r[...] = u0
        t0_r[...] = u0 * dinv

    return pl.pallas_call(
        body, out_shape=(_f32((N, 1)), _f32((N, D)), _f32((N, D)))
    )(degp, x, W0)


def _tc_b(s0, u0, dinv, b0, g0, be0, W1):
    N, D = u0.shape

    def body(s_r, u_r, dinv_r, b_r, g_r, be_r, w_r, h0_r, u1_r, t1_r):
        dinv = dinv_r[...]
        conv = dinv * (s_r[0] + s_r[1]) + dinv * dinv * u_r[...] + b_r[...]
        h0 = _bn_relu(conv, g_r[...], be_r[...])
        u1 = jnp.dot(h0, w_r[...], preferred_element_type=jnp.float32)
        h0_r[...] = h0
        u1_r[...] = u1
        t1_r[...] = u1 * dinv

    return pl.pallas_call(
        body, out_shape=(_f32((N, D)), _f32((N, D)), _f32((N, D)))
    )(s0, u0, dinv, b0, g0, be0, W1)


def _tc_c(s1, u1, h0, dinv, b1, g1, be1, Wt, bt):
    N, D = u1.shape

    def body(s_r, u_r, h0_r, dinv_r, b_r, g_r, be_r, wt_r, bt_r,
             h1_r, t_r, t2_r):
        dinv = dinv_r[...]
        conv = dinv * (s_r[0] + s_r[1]) + dinv * dinv * u_r[...] + b_r[...]
        h1 = _bn_relu(conv, g_r[...], be_r[...]) + h0_r[...]
        tt = jnp.dot(h1, wt_r[...], preferred_element_type=jnp.float32)
        t_r[...] = jax.nn.sigmoid(tt + bt_r[...])
        h1_r[...] = h1
        t2_r[...] = h1 * dinv

    return pl.pallas_call(
        body, out_shape=(_f32((N, D)), _f32((N, 1)), _f32((N, D)))
    )(s1, u1, h0, dinv, b1, g1, be1, Wt, bt)


def _tc_d(s2, h1, dinv, Wm, bm, Wv, bv, eps):
    N, D = h1.shape
    Z = Wm.shape[1]

    def body(s_r, h1_r, dinv_r, wm_r, bm_r, wv_r, bv_r, eps_r,
             qz_r, qm_r, qs_r):
        dinv = dinv_r[...]
        aggh = dinv * (s_r[0] + s_r[1]) + dinv * dinv * h1_r[...]
        q_m = jnp.dot(aggh, wm_r[...], preferred_element_type=jnp.float32) + bm_r[...]
        q_s = jnp.dot(aggh, wv_r[...], preferred_element_type=jnp.float32) + bv_r[...]
        std = jax.nn.softplus(q_s) + 1e-6
        qm_r[...] = q_m
        qs_r[...] = q_s
        qz_r[...] = q_m + std * eps_r[...]

    return pl.pallas_call(
        body, out_shape=(_f32((N, Z)), _f32((N, Z)), _f32((N, Z)))
    )(s2, h1, dinv, Wm, bm, Wv, bv, eps)


def kernel(x, edge_index, W0, b0, g0, be0, W1, b1, g1, be1,
           Wm, bm, Wv, bv, Wt, bt, eps):
    N, D = x.shape
    E = edge_index.shape[1]
    k_chunks = -(-E // (_NW * _CHUNK))
    if k_chunks % 2:
        k_chunks += 1
    e_pad = _NW * k_chunks * _CHUNK - E
    src = jnp.concatenate(
        [edge_index[0].astype(jnp.int32), jnp.zeros((e_pad,), jnp.int32)]
    )
    dst = jnp.concatenate(
        [edge_index[1].astype(jnp.int32), jnp.full((e_pad,), N, jnp.int32)]
    )

    ones_c = jnp.ones((_CHUNK, 128), jnp.float32)
    npad = ((N + 8 * _NS - 1) // (8 * _NS)) * (8 * _NS)
    zrows = jnp.zeros((npad // _NS, D), jnp.float32)

    deg_fn = _make_deg_kernel(k_chunks, N)
    agg_fn = _make_agg_kernel(k_chunks, N, D)

    degp = deg_fn(dst, ones_c, zrows)[:, :N, 0:1]
    dinv, u0, t0 = _tc_a(degp, x, W0)
    s0 = agg_fn(t0, src, dst, zrows)[:, :N]
    h0, u1, t1 = _tc_b(s0, u0, dinv, b0, g0, be0, W1)
    s1 = agg_fn(t1, src, dst, zrows)[:, :N]
    h1, t, t2 = _tc_c(s1, u1, h0, dinv, b1, g1, be1, Wt, bt)
    s2 = agg_fn(t2, src, dst, zrows)[:, :N]
    q_z, q_m, q_s = _tc_d(s2, h1, dinv, Wm, bm, Wv, bv, eps)
    return (q_z, q_m, q_s, t)


# speedup vs baseline: 3.2159x; 3.2159x over previous
import functools

import jax
import jax.numpy as jnp
from jax import lax
from jax.experimental import pallas as pl
from jax.experimental.pallas import tpu as pltpu
from jax.experimental.pallas import tpu_sc as plsc

_NC = 2
_NS = 16
_NW = _NC * _NS
_CHUNK = 128


def _f32(shape):
    return jax.ShapeDtypeStruct(shape, jnp.float32)


def _off(base, i):
    return pl.ds(pl.multiple_of(base + i * _CHUNK, 8), _CHUNK)


def _make_deg_kernel(K, N, W=128):
    npad = ((N + 8 * _NS - 1) // (8 * _NS)) * (8 * _NS)
    rpt = npad // _NS
    mesh = plsc.VectorSubcoreMesh(core_axis_name="c", subcore_axis_name="s")

    assert K % 2 == 0

    @functools.partial(
        pl.kernel,
        out_type=_f32((_NC, npad, W)),
        mesh=mesh,
        scratch_types=[
            pltpu.VMEM((_CHUNK,), jnp.int32),
            pltpu.VMEM((_CHUNK,), jnp.int32),
            pltpu.VMEM((_CHUNK, W), jnp.float32),
            pltpu.VMEM_SHARED((npad, W), jnp.float32),
            pltpu.SemaphoreType.DMA,
            pltpu.SemaphoreType.DMA,
        ],
    )
    def deg_kernel(dst_h, ones_h, zrows_h, out_h, di0, di1, ones_v, acc,
                   ss0, ss1):
        cid = lax.axis_index("c")
        sid = lax.axis_index("s")
        wid = sid * _NC + cid
        base = wid * (K * _CHUNK)
        pltpu.sync_copy(ones_h, ones_v)
        pltpu.sync_copy(zrows_h, acc.at[pl.ds(sid * rpt, rpt)])
        plsc.subcore_barrier()

        pltpu.sync_copy(dst_h.at[_off(base, 0)], di0)
        pltpu.async_copy(ones_v, acc.at[di0], ss0, add=True)

        def pair(j, carry):
            i0 = 2 * j
            pltpu.sync_copy(dst_h.at[_off(base, i0 + 1)], di1)
            pltpu.async_copy(ones_v, acc.at[di1], ss1, add=True)
            pltpu.make_async_copy(ones_v, acc.at[di0], ss0).wait()

            @pl.when(j < K // 2 - 1)
            def _():
                pltpu.sync_copy(dst_h.at[_off(base, i0 + 2)], di0)
                pltpu.async_copy(ones_v, acc.at[di0], ss0, add=True)

            pltpu.make_async_copy(ones_v, acc.at[di1], ss1).wait()
            return carry

        lax.fori_loop(0, K // 2, pair, 0)
        plsc.subcore_barrier()
        pltpu.sync_copy(
            acc.at[pl.ds(sid * rpt, rpt)],
            out_h.at[cid, pl.ds(sid * rpt, rpt)],
        )

    return deg_kernel


def _make_agg_kernel(K, N, D):
    npad = ((N + 8 * _NS - 1) // (8 * _NS)) * (8 * _NS)
    rpt = npad // _NS
    mesh = plsc.VectorSubcoreMesh(core_axis_name="c", subcore_axis_name="s")
    assert K % 2 == 0

    @functools.partial(
        pl.kernel,
        out_type=_f32((_NC, npad, D)),
        mesh=mesh,
        scratch_types=[
            pltpu.VMEM((_CHUNK,), jnp.int32),
            pltpu.VMEM((_CHUNK,), jnp.int32),
            pltpu.VMEM((_CHUNK,), jnp.int32),
            pltpu.VMEM((_CHUNK,), jnp.int32),
            pltpu.VMEM((_CHUNK, D), jnp.float32),
            pltpu.VMEM((_CHUNK, D), jnp.float32),
            pltpu.VMEM_SHARED((npad, D), jnp.float32),
            pltpu.SemaphoreType.DMA,
            pltpu.SemaphoreType.DMA,
        ],
    )
    def agg_kernel(table_h, src_h, dst_h, zrows_h, out_h,
                   si0, si1, di0, di1, rows0, rows1, acc, sg0, sg1):
        cid = lax.axis_index("c")
        sid = lax.axis_index("s")
        wid = sid * _NC + cid
        base = wid * (K * _CHUNK)
        pltpu.sync_copy(zrows_h, acc.at[pl.ds(sid * rpt, rpt)])
        plsc.subcore_barrier()

        pltpu.sync_copy(src_h.at[_off(base, 0)], si0)
        pltpu.sync_copy(dst_h.at[_off(base, 0)], di0)
        pltpu.async_copy(table_h.at[si0], rows0, sg0)

        def pair(j, carry):
            i0 = 2 * j
            pltpu.sync_copy(src_h.at[_off(base, i0 + 1)], si1)
            pltpu.sync_copy(dst_h.at[_off(base, i0 + 1)], di1)
            pltpu.async_copy(table_h.at[si1], rows1, sg1)
            pltpu.make_async_copy(table_h.at[si0], rows0, sg0).wait()
            pltpu.sync_copy(rows0, acc.at[di0], add=True)

            @pl.when(j < K // 2 - 1)
            def _():
                pltpu.sync_copy(src_h.at[_off(base, i0 + 2)], si0)
                pltpu.sync_copy(dst_h.at[_off(base, i0 + 2)], di0)
                pltpu.async_copy(table_h.at[si0], rows0, sg0)

            pltpu.make_async_copy(table_h.at[si1], rows1, sg1).wait()
            pltpu.sync_copy(rows1, acc.at[di1], add=True)
            return carry

        lax.fori_loop(0, K // 2, pair, 0)
        plsc.subcore_barrier()
        pltpu.sync_copy(
            acc.at[pl.ds(sid * rpt, rpt)],
            out_h.at[cid, pl.ds(sid * rpt, rpt)],
        )

    return agg_kernel


def _bn_relu(conv, g, be):
    m = jnp.mean(conv, axis=0)
    v = jnp.var(conv, axis=0)
    return jnp.maximum((conv - m) / jnp.sqrt(v + 1e-5) * g + be, 0.0)


def _tc_a(degp, x, W0):
    N, D = x.shape

    def body(degp_r, x_r, w_r, dinv_r, u0_r, t0_r):
        deg = degp_r[0] + degp_r[1] + 1.0
        dinv = lax.rsqrt(deg)
        u0 = jnp.dot(x_r[...], w_r[...], preferred_element_type=jnp.float32)
        dinv_r[...] = dinv
        u0_r[...] = u0
        t0_r[...] = u0 * dinv

    return pl.pallas_call(
        body, out_shape=(_f32((N, 1)), _f32((N, D)), _f32((N, D)))
    )(degp, x, W0)


def _tc_b(s0, u0, dinv, b0, g0, be0, W1):
    N, D = u0.shape

    def body(s_r, u_r, dinv_r, b_r, g_r, be_r, w_r, h0_r, u1_r, t1_r):
        dinv = dinv_r[...]
        conv = dinv * (s_r[0] + s_r[1]) + dinv * dinv * u_r[...] + b_r[...]
        h0 = _bn_relu(conv, g_r[...], be_r[...])
        u1 = jnp.dot(h0, w_r[...], preferred_element_type=jnp.float32)
        h0_r[...] = h0
        u1_r[...] = u1
        t1_r[...] = u1 * dinv

    return pl.pallas_call(
        body, out_shape=(_f32((N, D)), _f32((N, D)), _f32((N, D)))
    )(s0, u0, dinv, b0, g0, be0, W1)


def _tc_c(s1, u1, h0, dinv, b1, g1, be1, Wt, bt):
    N, D = u1.shape

    def body(s_r, u_r, h0_r, dinv_r, b_r, g_r, be_r, wt_r, bt_r,
             h1_r, t_r, t2_r):
        dinv = dinv_r[...]
        conv = dinv * (s_r[0] + s_r[1]) + dinv * dinv * u_r[...] + b_r[...]
        h1 = _bn_relu(conv, g_r[...], be_r[...]) + h0_r[...]
        tt = jnp.dot(h1, wt_r[...], preferred_element_type=jnp.float32)
        t_r[...] = jax.nn.sigmoid(tt + bt_r[...])
        h1_r[...] = h1
        t2_r[...] = h1 * dinv

    return pl.pallas_call(
        body, out_shape=(_f32((N, D)), _f32((N, 1)), _f32((N, D)))
    )(s1, u1, h0, dinv, b1, g1, be1, Wt, bt)


def _tc_d(s2, h1, dinv, Wm, bm, Wv, bv, eps):
    N, D = h1.shape
    Z = Wm.shape[1]

    def body(s_r, h1_r, dinv_r, wm_r, bm_r, wv_r, bv_r, eps_r,
             qz_r, qm_r, qs_r):
        dinv = dinv_r[...]
        aggh = dinv * (s_r[0] + s_r[1]) + dinv * dinv * h1_r[...]
        q_m = jnp.dot(aggh, wm_r[...], preferred_element_type=jnp.float32) + bm_r[...]
        q_s = jnp.dot(aggh, wv_r[...], preferred_element_type=jnp.float32) + bv_r[...]
        std = jax.nn.softplus(q_s) + 1e-6
        qm_r[...] = q_m
        qs_r[...] = q_s
        qz_r[...] = q_m + std * eps_r[...]

    return pl.pallas_call(
        body, out_shape=(_f32((N, Z)), _f32((N, Z)), _f32((N, Z)))
    )(s2, h1, dinv, Wm, bm, Wv, bv, eps)


def kernel(x, edge_index, W0, b0, g0, be0, W1, b1, g1, be1,
           Wm, bm, Wv, bv, Wt, bt, eps):
    N, D = x.shape
    E = edge_index.shape[1]
    k_chunks = -(-E // (_NW * _CHUNK))
    if k_chunks % 2:
        k_chunks += 1
    e_pad = _NW * k_chunks * _CHUNK - E
    npad = ((N + 8 * _NS - 1) // (8 * _NS)) * (8 * _NS)
    pad_ids = jnp.arange(e_pad, dtype=jnp.int32)
    src = jnp.concatenate(
        [edge_index[0].astype(jnp.int32), pad_ids % N]
    )
    dst = jnp.concatenate(
        [edge_index[1].astype(jnp.int32), N + pad_ids % (npad - N)]
    )

    ones_c = jnp.ones((_CHUNK, 128), jnp.float32)
    zrows = jnp.zeros((npad // _NS, D), jnp.float32)

    deg_fn = _make_deg_kernel(k_chunks, N)
    agg_fn = _make_agg_kernel(k_chunks, N, D)

    degp = deg_fn(dst, ones_c, zrows)[:, :N, 0:1]
    dinv, u0, t0 = _tc_a(degp, x, W0)
    s0 = agg_fn(t0, src, dst, zrows)[:, :N]
    h0, u1, t1 = _tc_b(s0, u0, dinv, b0, g0, be0, W1)
    s1 = agg_fn(t1, src, dst, zrows)[:, :N]
    h1, t, t2 = _tc_c(s1, u1, h0, dinv, b1, g1, be1, Wt, bt)
    s2 = agg_fn(t2, src, dst, zrows)[:, :N]
    q_z, q_m, q_s = _tc_d(s2, h1, dinv, Wm, bm, Wv, bv, eps)
    return (q_z, q_m, q_s, t)


# speedup vs baseline: 3.3483x; 1.0412x over previous
import functools

import jax
import jax.numpy as jnp
from jax import lax
from jax.experimental import pallas as pl
from jax.experimental.pallas import tpu as pltpu
from jax.experimental.pallas import tpu_sc as plsc

_NC = 2
_NS = 16
_NW = _NC * _NS
_CHUNK = 128


def _f32(shape):
    return jax.ShapeDtypeStruct(shape, jnp.float32)


def _off(base, i):
    return pl.ds(pl.multiple_of(base + i * _CHUNK, 8), _CHUNK)


def _make_deg_kernel(K, N, W=128):
    npad = ((N + 8 * _NS - 1) // (8 * _NS)) * (8 * _NS)
    rpt = npad // _NS
    mesh = plsc.VectorSubcoreMesh(core_axis_name="c", subcore_axis_name="s")

    assert K % 2 == 0

    @functools.partial(
        pl.kernel,
        out_type=_f32((_NC, npad, W)),
        mesh=mesh,
        scratch_types=[
            pltpu.VMEM((_CHUNK,), jnp.int32),
            pltpu.VMEM((_CHUNK,), jnp.int32),
            pltpu.VMEM((_CHUNK, W), jnp.float32),
            pltpu.VMEM_SHARED((npad, W), jnp.float32),
            pltpu.SemaphoreType.DMA,
            pltpu.SemaphoreType.DMA,
        ],
    )
    def deg_kernel(dst_h, ones_h, zrows_h, out_h, di0, di1, ones_v, acc,
                   ss0, ss1):
        cid = lax.axis_index("c")
        sid = lax.axis_index("s")
        wid = sid * _NC + cid
        base = wid * (K * _CHUNK)
        pltpu.sync_copy(ones_h, ones_v)
        pltpu.sync_copy(zrows_h, acc.at[pl.ds(sid * rpt, rpt)])
        plsc.subcore_barrier()

        pltpu.sync_copy(dst_h.at[_off(base, 0)], di0)
        pltpu.async_copy(ones_v, acc.at[di0], ss0, add=True)

        def pair(j, carry):
            i0 = 2 * j
            pltpu.sync_copy(dst_h.at[_off(base, i0 + 1)], di1)
            pltpu.async_copy(ones_v, acc.at[di1], ss1, add=True)
            pltpu.make_async_copy(ones_v, acc.at[di0], ss0).wait()

            @pl.when(j < K // 2 - 1)
            def _():
                pltpu.sync_copy(dst_h.at[_off(base, i0 + 2)], di0)
                pltpu.async_copy(ones_v, acc.at[di0], ss0, add=True)

            pltpu.make_async_copy(ones_v, acc.at[di1], ss1).wait()
            return carry

        lax.fori_loop(0, K // 2, pair, 0)
        plsc.subcore_barrier()
        pltpu.sync_copy(
            acc.at[pl.ds(sid * rpt, rpt)],
            out_h.at[cid, pl.ds(sid * rpt, rpt)],
        )

    return deg_kernel


def _make_agg_kernel(K, N, D):
    npad = ((N + 8 * _NS - 1) // (8 * _NS)) * (8 * _NS)
    rpt = npad // _NS
    mesh = plsc.VectorSubcoreMesh(core_axis_name="c", subcore_axis_name="s")
    assert K % 2 == 0

    @functools.partial(
        pl.kernel,
        out_type=_f32((_NC, npad, D)),
        mesh=mesh,
        scratch_types=[
            pltpu.VMEM((_CHUNK,), jnp.int32),
            pltpu.VMEM((_CHUNK,), jnp.int32),
            pltpu.VMEM((_CHUNK,), jnp.int32),
            pltpu.VMEM((_CHUNK,), jnp.int32),
            pltpu.VMEM((_CHUNK, D), jnp.float32),
            pltpu.VMEM((_CHUNK, D), jnp.float32),
            pltpu.VMEM_SHARED((npad, D), jnp.float32),
            pltpu.SemaphoreType.DMA,
            pltpu.SemaphoreType.DMA,
        ],
    )
    def agg_kernel(table_h, src_h, dst_h, zrows_h, out_h,
                   si0, si1, di0, di1, rows0, rows1, acc, sg0, sg1):
        cid = lax.axis_index("c")
        sid = lax.axis_index("s")
        wid = sid * _NC + cid
        base = wid * (K * _CHUNK)
        pltpu.sync_copy(zrows_h, acc.at[pl.ds(sid * rpt, rpt)])
        plsc.subcore_barrier()

        pltpu.sync_copy(src_h.at[_off(base, 0)], si0)
        pltpu.sync_copy(dst_h.at[_off(base, 0)], di0)
        pltpu.async_copy(table_h.at[si0], rows0, sg0)

        def pair(j, carry):
            i0 = 2 * j
            pltpu.sync_copy(src_h.at[_off(base, i0 + 1)], si1)
            pltpu.sync_copy(dst_h.at[_off(base, i0 + 1)], di1)
            pltpu.async_copy(table_h.at[si1], rows1, sg1)
            pltpu.make_async_copy(table_h.at[si0], rows0, sg0).wait()
            pltpu.sync_copy(rows0, acc.at[di0], add=True)

            @pl.when(j < K // 2 - 1)
            def _():
                pltpu.sync_copy(src_h.at[_off(base, i0 + 2)], si0)
                pltpu.sync_copy(dst_h.at[_off(base, i0 + 2)], di0)
                pltpu.async_copy(table_h.at[si0], rows0, sg0)

            pltpu.make_async_copy(table_h.at[si1], rows1, sg1).wait()
            pltpu.sync_copy(rows1, acc.at[di1], add=True)
            return carry

        lax.fori_loop(0, K // 2, pair, 0)
        plsc.subcore_barrier()
        pltpu.sync_copy(
            acc.at[pl.ds(sid * rpt, rpt)],
            out_h.at[cid, pl.ds(sid * rpt, rpt)],
        )

    return agg_kernel


def _bn_relu(conv, g, be):
    m = jnp.mean(conv, axis=0)
    v = jnp.var(conv, axis=0)
    return jnp.maximum((conv - m) / jnp.sqrt(v + 1e-5) * g + be, 0.0)


def _tc_a(degp, x, W0):
    N, D = x.shape

    def body(degp_r, x_r, w_r, dinv_r, u0_r, t0_r):
        deg = degp_r[0, :N, 0:1] + degp_r[1, :N, 0:1] + 1.0
        dinv = lax.rsqrt(deg)
        u0 = jnp.dot(x_r[...], w_r[...], preferred_element_type=jnp.float32)
        dinv_r[...] = dinv
        u0_r[...] = u0
        t0_r[...] = u0 * dinv

    return pl.pallas_call(
        body, out_shape=(_f32((N, 1)), _f32((N, D)), _f32((N, D)))
    )(degp, x, W0)


def _tc_b(s0, u0, dinv, b0, g0, be0, W1):
    N, D = u0.shape

    def body(s_r, u_r, dinv_r, b_r, g_r, be_r, w_r, h0_r, u1_r, t1_r):
        dinv = dinv_r[...]
        conv = dinv * (s_r[0, :N] + s_r[1, :N]) + dinv * dinv * u_r[...] + b_r[...]
        h0 = _bn_relu(conv, g_r[...], be_r[...])
        u1 = jnp.dot(h0, w_r[...], preferred_element_type=jnp.float32)
        h0_r[...] = h0
        u1_r[...] = u1
        t1_r[...] = u1 * dinv

    return pl.pallas_call(
        body, out_shape=(_f32((N, D)), _f32((N, D)), _f32((N, D)))
    )(s0, u0, dinv, b0, g0, be0, W1)


def _tc_c(s1, u1, h0, dinv, b1, g1, be1, Wt, bt):
    N, D = u1.shape

    def body(s_r, u_r, h0_r, dinv_r, b_r, g_r, be_r, wt_r, bt_r,
             h1_r, t_r, t2_r):
        dinv = dinv_r[...]
        conv = dinv * (s_r[0, :N] + s_r[1, :N]) + dinv * dinv * u_r[...] + b_r[...]
        h1 = _bn_relu(conv, g_r[...], be_r[...]) + h0_r[...]
        tt = jnp.dot(h1, wt_r[...], preferred_element_type=jnp.float32)
        t_r[...] = jax.nn.sigmoid(tt + bt_r[...])
        h1_r[...] = h1
        t2_r[...] = h1 * dinv

    return pl.pallas_call(
        body, out_shape=(_f32((N, D)), _f32((N, 1)), _f32((N, D)))
    )(s1, u1, h0, dinv, b1, g1, be1, Wt, bt)


def _tc_d(s2, h1, dinv, Wm, bm, Wv, bv, eps):
    N, D = h1.shape
    Z = Wm.shape[1]

    def body(s_r, h1_r, dinv_r, wm_r, bm_r, wv_r, bv_r, eps_r,
             qz_r, qm_r, qs_r):
        dinv = dinv_r[...]
        aggh = dinv * (s_r[0, :N] + s_r[1, :N]) + dinv * dinv * h1_r[...]
        q_m = jnp.dot(aggh, wm_r[...], preferred_element_type=jnp.float32) + bm_r[...]
        q_s = jnp.dot(aggh, wv_r[...], preferred_element_type=jnp.float32) + bv_r[...]
        std = jax.nn.softplus(q_s) + 1e-6
        qm_r[...] = q_m
        qs_r[...] = q_s
        qz_r[...] = q_m + std * eps_r[...]

    return pl.pallas_call(
        body, out_shape=(_f32((N, Z)), _f32((N, Z)), _f32((N, Z)))
    )(s2, h1, dinv, Wm, bm, Wv, bv, eps)


def kernel(x, edge_index, W0, b0, g0, be0, W1, b1, g1, be1,
           Wm, bm, Wv, bv, Wt, bt, eps):
    N, D = x.shape
    E = edge_index.shape[1]
    k_chunks = -(-E // (_NW * _CHUNK))
    if k_chunks % 2:
        k_chunks += 1
    e_pad = _NW * k_chunks * _CHUNK - E
    npad = ((N + 8 * _NS - 1) // (8 * _NS)) * (8 * _NS)
    pad_ids = jnp.arange(e_pad, dtype=jnp.int32)
    src = jnp.concatenate(
        [edge_index[0].astype(jnp.int32), pad_ids % N]
    )
    dst = jnp.concatenate(
        [edge_index[1].astype(jnp.int32), N + pad_ids % (npad - N)]
    )

    ones_c = jnp.ones((_CHUNK, 128), jnp.float32)
    zrows = jnp.zeros((npad // _NS, D), jnp.float32)

    deg_fn = _make_deg_kernel(k_chunks, N)
    agg_fn = _make_agg_kernel(k_chunks, N, D)

    degp = deg_fn(dst, ones_c, zrows)
    dinv, u0, t0 = _tc_a(degp, x, W0)
    s0 = agg_fn(t0, src, dst, zrows)
    h0, u1, t1 = _tc_b(s0, u0, dinv, b0, g0, be0, W1)
    s1 = agg_fn(t1, src, dst, zrows)
    h1, t, t2 = _tc_c(s1, u1, h0, dinv, b1, g1, be1, Wt, bt)
    s2 = agg_fn(t2, src, dst, zrows)
    q_z, q_m, q_s = _tc_d(s2, h1, dinv, Wm, bm, Wv, bv, eps)
    return (q_z, q_m, q_s, t)


# speedup vs baseline: 3.8592x; 1.1526x over previous
import functools

import jax
import jax.numpy as jnp
from jax import lax
from jax.experimental import pallas as pl
from jax.experimental.pallas import tpu as pltpu
from jax.experimental.pallas import tpu_sc as plsc

_NC = 2
_NS = 16
_NW = _NC * _NS
_CHUNK = 128


def _f32(shape):
    return jax.ShapeDtypeStruct(shape, jnp.float32)


def _off(base, i):
    return pl.ds(pl.multiple_of(base + i * _CHUNK, 8), _CHUNK)


def _make_deg_kernel(K, N, W=128):
    npad = ((N + 8 * _NS - 1) // (8 * _NS)) * (8 * _NS)
    rpt = npad // _NS
    mesh = plsc.VectorSubcoreMesh(core_axis_name="c", subcore_axis_name="s")

    assert K % 2 == 1

    @functools.partial(
        pl.kernel,
        out_type=_f32((_NC, npad, W)),
        mesh=mesh,
        scratch_types=[
            pltpu.VMEM((_CHUNK,), jnp.int32),
            pltpu.VMEM((_CHUNK,), jnp.int32),
            pltpu.VMEM((_CHUNK, W), jnp.float32),
            pltpu.VMEM_SHARED((npad, W), jnp.float32),
            pltpu.SemaphoreType.DMA,
            pltpu.SemaphoreType.DMA,
        ],
    )
    def deg_kernel(dst_h, ones_h, zrows_h, out_h, di0, di1, ones_v, acc,
                   ss0, ss1):
        cid = lax.axis_index("c")
        sid = lax.axis_index("s")
        wid = sid * _NC + cid
        base = wid * (K * _CHUNK)
        pltpu.sync_copy(ones_h, ones_v)
        pltpu.sync_copy(zrows_h, acc.at[pl.ds(sid * rpt, rpt)])
        plsc.subcore_barrier()

        pltpu.sync_copy(dst_h.at[_off(base, 0)], di0)
        pltpu.async_copy(ones_v, acc.at[di0], ss0, add=True)

        def pair(j, carry):
            a = 2 * j + 1
            pltpu.sync_copy(dst_h.at[_off(base, a)], di1)
            pltpu.async_copy(ones_v, acc.at[di1], ss1, add=True)
            pltpu.make_async_copy(ones_v, acc.at[di0], ss0).wait()
            pltpu.sync_copy(dst_h.at[_off(base, a + 1)], di0)
            pltpu.async_copy(ones_v, acc.at[di0], ss0, add=True)
            pltpu.make_async_copy(ones_v, acc.at[di1], ss1).wait()
            return carry

        lax.fori_loop(0, (K - 1) // 2, pair, 0)
        pltpu.make_async_copy(ones_v, acc.at[di0], ss0).wait()
        plsc.subcore_barrier()
        pltpu.sync_copy(
            acc.at[pl.ds(sid * rpt, rpt)],
            out_h.at[cid, pl.ds(sid * rpt, rpt)],
        )

    return deg_kernel


def _make_agg_kernel(K, N, D):
    npad = ((N + 8 * _NS - 1) // (8 * _NS)) * (8 * _NS)
    rpt = npad // _NS
    mesh = plsc.VectorSubcoreMesh(core_axis_name="c", subcore_axis_name="s")
    assert K % 3 == 0

    @functools.partial(
        pl.kernel,
        out_type=_f32((_NC, npad, D)),
        mesh=mesh,
        scratch_types=[
            pltpu.VMEM((_CHUNK,), jnp.int32),
            pltpu.VMEM((_CHUNK,), jnp.int32),
            pltpu.VMEM((_CHUNK,), jnp.int32),
            pltpu.VMEM((_CHUNK,), jnp.int32),
            pltpu.VMEM((_CHUNK,), jnp.int32),
            pltpu.VMEM((_CHUNK,), jnp.int32),
            pltpu.VMEM((_CHUNK, D), jnp.float32),
            pltpu.VMEM((_CHUNK, D), jnp.float32),
            pltpu.VMEM((_CHUNK, D), jnp.float32),
            pltpu.VMEM_SHARED((npad, D), jnp.float32),
            pltpu.SemaphoreType.DMA,
            pltpu.SemaphoreType.DMA,
            pltpu.SemaphoreType.DMA,
            pltpu.SemaphoreType.DMA,
            pltpu.SemaphoreType.DMA,
            pltpu.SemaphoreType.DMA,
        ],
    )
    def agg_kernel(table_h, src_h, dst_h, zrows_h, out_h,
                   si0, si1, si2, di0, di1, di2, rows0, rows1, rows2, acc,
                   sg0, sg1, sg2, ss0, ss1, ss2):
        cid = lax.axis_index("c")
        sid = lax.axis_index("s")
        wid = sid * _NC + cid
        base = wid * (K * _CHUNK)
        pltpu.sync_copy(zrows_h, acc.at[pl.ds(sid * rpt, rpt)])
        plsc.subcore_barrier()

        pltpu.sync_copy(src_h.at[_off(base, 0)], si0)
        pltpu.sync_copy(dst_h.at[_off(base, 0)], di0)
        pltpu.async_copy(table_h.at[si0], rows0, sg0)
        pltpu.sync_copy(src_h.at[_off(base, 1)], si1)
        pltpu.sync_copy(dst_h.at[_off(base, 1)], di1)
        pltpu.async_copy(table_h.at[si1], rows1, sg1)

        k3 = K // 3

        def ring(j, carry):
            c = 3 * j
            pltpu.make_async_copy(table_h.at[si0], rows0, sg0).wait()
            pltpu.async_copy(rows0, acc.at[di0], ss0, add=True)

            @pl.when(j > 0)
            def _():
                pltpu.make_async_copy(rows2, acc.at[di2], ss2).wait()

            pltpu.sync_copy(src_h.at[_off(base, c + 2)], si2)
            pltpu.sync_copy(dst_h.at[_off(base, c + 2)], di2)
            pltpu.async_copy(table_h.at[si2], rows2, sg2)

            pltpu.make_async_copy(table_h.at[si1], rows1, sg1).wait()
            pltpu.async_copy(rows1, acc.at[di1], ss1, add=True)

            @pl.when(j < k3 - 1)
            def _():
                pltpu.make_async_copy(rows0, acc.at[di0], ss0).wait()
                pltpu.sync_copy(src_h.at[_off(base, c + 3)], si0)
                pltpu.sync_copy(dst_h.at[_off(base, c + 3)], di0)
                pltpu.async_copy(table_h.at[si0], rows0, sg0)

            pltpu.make_async_copy(table_h.at[si2], rows2, sg2).wait()
            pltpu.async_copy(rows2, acc.at[di2], ss2, add=True)

            @pl.when(j < k3 - 1)
            def _():
                pltpu.make_async_copy(rows1, acc.at[di1], ss1).wait()
                pltpu.sync_copy(src_h.at[_off(base, c + 4)], si1)
                pltpu.sync_copy(dst_h.at[_off(base, c + 4)], di1)
                pltpu.async_copy(table_h.at[si1], rows1, sg1)

            return carry

        lax.fori_loop(0, k3, ring, 0)
        pltpu.make_async_copy(rows0, acc.at[di0], ss0).wait()
        pltpu.make_async_copy(rows1, acc.at[di1], ss1).wait()
        pltpu.make_async_copy(rows2, acc.at[di2], ss2).wait()
        plsc.subcore_barrier()
        pltpu.sync_copy(
            acc.at[pl.ds(sid * rpt, rpt)],
            out_h.at[cid, pl.ds(sid * rpt, rpt)],
        )

    return agg_kernel


def _bn_relu(conv, g, be):
    m = jnp.mean(conv, axis=0)
    v = jnp.var(conv, axis=0)
    return jnp.maximum((conv - m) / jnp.sqrt(v + 1e-5) * g + be, 0.0)


def _tc_a(degp, x, W0):
    N, D = x.shape

    def body(degp_r, x_r, w_r, dinv_r, u0_r, t0_r):
        deg = degp_r[0, :N, 0:1] + degp_r[1, :N, 0:1] + 1.0
        dinv = lax.rsqrt(deg)
        u0 = jnp.dot(x_r[...], w_r[...], preferred_element_type=jnp.float32)
        dinv_r[...] = dinv
        u0_r[...] = u0
        t0_r[...] = u0 * dinv

    return pl.pallas_call(
        body, out_shape=(_f32((N, 1)), _f32((N, D)), _f32((N, D)))
    )(degp, x, W0)


def _tc_b(s0, u0, dinv, b0, g0, be0, W1):
    N, D = u0.shape

    def body(s_r, u_r, dinv_r, b_r, g_r, be_r, w_r, h0_r, u1_r, t1_r):
        dinv = dinv_r[...]
        conv = dinv * (s_r[0, :N] + s_r[1, :N]) + dinv * dinv * u_r[...] + b_r[...]
        h0 = _bn_relu(conv, g_r[...], be_r[...])
        u1 = jnp.dot(h0, w_r[...], preferred_element_type=jnp.float32)
        h0_r[...] = h0
        u1_r[...] = u1
        t1_r[...] = u1 * dinv

    return pl.pallas_call(
        body, out_shape=(_f32((N, D)), _f32((N, D)), _f32((N, D)))
    )(s0, u0, dinv, b0, g0, be0, W1)


def _tc_c(s1, u1, h0, dinv, b1, g1, be1, Wt, bt):
    N, D = u1.shape

    def body(s_r, u_r, h0_r, dinv_r, b_r, g_r, be_r, wt_r, bt_r,
             h1_r, t_r, t2_r):
        dinv = dinv_r[...]
        conv = dinv * (s_r[0, :N] + s_r[1, :N]) + dinv * dinv * u_r[...] + b_r[...]
        h1 = _bn_relu(conv, g_r[...], be_r[...]) + h0_r[...]
        tt = jnp.dot(h1, wt_r[...], preferred_element_type=jnp.float32)
        t_r[...] = jax.nn.sigmoid(tt + bt_r[...])
        h1_r[...] = h1
        t2_r[...] = h1 * dinv

    return pl.pallas_call(
        body, out_shape=(_f32((N, D)), _f32((N, 1)), _f32((N, D)))
    )(s1, u1, h0, dinv, b1, g1, be1, Wt, bt)


def _tc_d(s2, h1, dinv, Wm, bm, Wv, bv, eps):
    N, D = h1.shape
    Z = Wm.shape[1]

    def body(s_r, h1_r, dinv_r, wm_r, bm_r, wv_r, bv_r, eps_r,
             qz_r, qm_r, qs_r):
        dinv = dinv_r[...]
        aggh = dinv * (s_r[0, :N] + s_r[1, :N]) + dinv * dinv * h1_r[...]
        q_m = jnp.dot(aggh, wm_r[...], preferred_element_type=jnp.float32) + bm_r[...]
        q_s = jnp.dot(aggh, wv_r[...], preferred_element_type=jnp.float32) + bv_r[...]
        std = jax.nn.softplus(q_s) + 1e-6
        qm_r[...] = q_m
        qs_r[...] = q_s
        qz_r[...] = q_m + std * eps_r[...]

    return pl.pallas_call(
        body, out_shape=(_f32((N, Z)), _f32((N, Z)), _f32((N, Z)))
    )(s2, h1, dinv, Wm, bm, Wv, bv, eps)


def kernel(x, edge_index, W0, b0, g0, be0, W1, b1, g1, be1,
           Wm, bm, Wv, bv, Wt, bt, eps):
    N, D = x.shape
    E = edge_index.shape[1]
    k_chunks = -(-E // (_NW * _CHUNK))
    while k_chunks % 3 or k_chunks % 2 == 0:
        k_chunks += 1
    e_pad = _NW * k_chunks * _CHUNK - E
    npad = ((N + 8 * _NS - 1) // (8 * _NS)) * (8 * _NS)
    pad_ids = jnp.arange(e_pad, dtype=jnp.int32)
    src = jnp.concatenate(
        [edge_index[0].astype(jnp.int32), pad_ids % N]
    )
    dst = jnp.concatenate(
        [edge_index[1].astype(jnp.int32), N + pad_ids % (npad - N)]
    )

    ones_c = jnp.ones((_CHUNK, 128), jnp.float32)
    zrows = jnp.zeros((npad // _NS, D), jnp.float32)

    deg_fn = _make_deg_kernel(k_chunks, N)
    agg_fn = _make_agg_kernel(k_chunks, N, D)

    degp = deg_fn(dst, ones_c, zrows)
    dinv, u0, t0 = _tc_a(degp, x, W0)
    s0 = agg_fn(t0, src, dst, zrows)
    h0, u1, t1 = _tc_b(s0, u0, dinv, b0, g0, be0, W1)
    s1 = agg_fn(t1, src, dst, zrows)
    h1, t, t2 = _tc_c(s1, u1, h0, dinv, b1, g1, be1, Wt, bt)
    s2 = agg_fn(t2, src, dst, zrows)
    q_z, q_m, q_s = _tc_d(s2, h1, dinv, Wm, bm, Wv, bv, eps)
    return (q_z, q_m, q_s, t)
